# baseline (speedup 1.0000x reference)
import functools

import jax
import jax.numpy as jnp
from jax import lax
from jax.experimental import pallas as pl
from jax.experimental.pallas import tpu as pltpu
from jax.experimental.pallas import tpu_sc as plsc

N = 10000
K = 32
F = 128
KF = K * F
TOTAL = N * K

NC = 2
NS = 16
NW = NC * NS
PER_W = TOTAL // NW
CH = 128
NFULL = PER_W // CH
TAIL = PER_W - NFULL * CH
NB = 3


def _sc_gather_kernel(table_hbm, idx_hbm, out_hbm, idx_v, rows_v, tail_v,
                      gsem, wsem, tsem):
    wid = lax.axis_index("s") * NC + lax.axis_index("c")
    base = pl.multiple_of(wid * PER_W, 16)
    pltpu.sync_copy(idx_hbm.at[pl.ds(base, PER_W)], idx_v)
    toff = NFULL * CH
    tg = pltpu.make_async_copy(
        table_hbm.at[idx_v.at[pl.ds(toff, TAIL)]], tail_v, tsem)
    tg.start()

    def group(i, carry):
        gathers = []
        for b in range(NB):
            off = pl.multiple_of((i * NB + b) * CH, CH)
            @pl.when(i >= 1)
            def _(off=off, b=b):
                prev = pl.multiple_of(off - NB * CH, CH)
                pltpu.make_async_copy(
                    rows_v.at[b], out_hbm.at[pl.ds(base + prev, CH)],
                    wsem.at[b]).wait()
            g = pltpu.make_async_copy(
                table_hbm.at[idx_v.at[pl.ds(off, CH)]], rows_v.at[b],
                gsem.at[b])
            g.start()
            gathers.append((off, b, g))
        for off, b, g in gathers:
            g.wait()
            pltpu.make_async_copy(
                rows_v.at[b], out_hbm.at[pl.ds(base + off, CH)],
                wsem.at[b]).start()
        return carry

    lax.fori_loop(0, NFULL // NB, group, 0, unroll=False)

    tg.wait()
    pltpu.sync_copy(tail_v, out_hbm.at[pl.ds(base + toff, TAIL)])
    for b in range(NB):
        off = (NFULL - NB + b) * CH
        pltpu.make_async_copy(
            rows_v.at[b], out_hbm.at[pl.ds(base + off, CH)], wsem.at[b]).wait()


def _sc_gather(table, idx_flat):
    mesh = plsc.VectorSubcoreMesh(core_axis_name="c", subcore_axis_name="s")
    return pl.kernel(
        _sc_gather_kernel,
        out_type=jax.ShapeDtypeStruct((TOTAL, F), jnp.float32),
        mesh=mesh,
        scratch_types=[
            pltpu.VMEM((PER_W,), jnp.int32),
            pltpu.VMEM((NB, CH, F), jnp.float32),
            pltpu.VMEM((TAIL, F), jnp.float32),
            pltpu.SemaphoreType.DMA((NB,)),
            pltpu.SemaphoreType.DMA((NB,)),
            pltpu.SemaphoreType.DMA,
        ],
    )(table, idx_flat)


BN = 1000
BL = 2000


def _ln_silu_body(x_ref, g_ref, b_ref, o_ref):
    x = x_ref[...]
    mu = jnp.mean(x, axis=-1, keepdims=True)
    var = jnp.mean((x - mu) ** 2, axis=-1, keepdims=True)
    t = (x - mu) / jnp.sqrt(var + 1e-5) * g_ref[...] + b_ref[...]
    o_ref[...] = t * jax.nn.sigmoid(t)


def _ln_silu(x, g, b):
    return pl.pallas_call(
        _ln_silu_body,
        grid=(N // BL,),
        in_specs=[
            pl.BlockSpec((BL, F), lambda i: (i, 0)),
            pl.BlockSpec((1, F), lambda i: (0, 0)),
            pl.BlockSpec((1, F), lambda i: (0, 0)),
        ],
        out_specs=pl.BlockSpec((BL, F), lambda i: (i, 0)),
        out_shape=jax.ShapeDtypeStruct((N, F), jnp.float32),
    )(x, g.reshape(1, F), b.reshape(1, F))


def _nh_dot(g_ref, w_ref):
    return jnp.dot(g_ref[...].reshape(BN, KF), w_ref[...].reshape(KF, F),
                   preferred_element_type=jnp.float32)


def _mm_ln_silu_body(g_ref, w_ref, b_ref, lg_ref, lb_ref, o_ref):
    y = _nh_dot(g_ref, w_ref) + b_ref[...]
    mu = jnp.mean(y, axis=-1, keepdims=True)
    var = jnp.mean((y - mu) ** 2, axis=-1, keepdims=True)
    t = (y - mu) / jnp.sqrt(var + 1e-5) * lg_ref[...] + lb_ref[...]
    o_ref[...] = t * jax.nn.sigmoid(t)


def _mm_ln_silu(gath, w, b, lg, lb):
    return pl.pallas_call(
        _mm_ln_silu_body,
        grid=(N // BN,),
        in_specs=[
            pl.BlockSpec((BN, K, F), lambda i: (i, 0, 0)),
            pl.BlockSpec((K, F, F), lambda i: (0, 0, 0)),
            pl.BlockSpec((1, F), lambda i: (0, 0)),
            pl.BlockSpec((1, F), lambda i: (0, 0)),
            pl.BlockSpec((1, F), lambda i: (0, 0)),
        ],
        out_specs=pl.BlockSpec((BN, F), lambda i: (i, 0)),
        out_shape=jax.ShapeDtypeStruct((N, F), jnp.float32),
    )(gath, w, b.reshape(1, F), lg.reshape(1, F), lb.reshape(1, F))


def _mm_res_body(g_ref, w_ref, b_ref, x_ref, o_ref):
    o_ref[...] = _nh_dot(g_ref, w_ref) + b_ref[...] + x_ref[...]


def _mm_res(gath, w, b, x):
    return pl.pallas_call(
        _mm_res_body,
        grid=(N // BN,),
        in_specs=[
            pl.BlockSpec((BN, K, F), lambda i: (i, 0, 0)),
            pl.BlockSpec((K, F, F), lambda i: (0, 0, 0)),
            pl.BlockSpec((1, F), lambda i: (0, 0)),
            pl.BlockSpec((BN, F), lambda i: (i, 0)),
        ],
        out_specs=pl.BlockSpec((BN, F), lambda i: (i, 0)),
        out_shape=jax.ShapeDtypeStruct((N, F), jnp.float32),
    )(gath, w, b.reshape(1, F), x)


def kernel(x, adjc, ln1_g, ln1_b, w1, b1, ln2_g, ln2_b, w2, b2):
    idx_flat = adjc.reshape(TOTAL)
    w1r = w1.reshape(K, F, F)
    w2r = w2.reshape(K, F, F)
    h1 = _ln_silu(x, ln1_g, ln1_b)
    g1 = _sc_gather(h1, idx_flat)
    h2 = _mm_ln_silu(g1.reshape(N, K, F), w1r, b1, ln2_g, ln2_b)
    g2 = _sc_gather(h2, idx_flat)
    return _mm_res(g2.reshape(N, K, F), w2r, b2, x)


# speedup vs baseline: 1.0098x; 1.0098x over previous
import functools

import jax
import jax.numpy as jnp
from jax import lax
from jax.experimental import pallas as pl
from jax.experimental.pallas import tpu as pltpu
from jax.experimental.pallas import tpu_sc as plsc

N = 10000
K = 32
F = 128
KF = K * F
TOTAL = N * K

NC = 2
NS = 16
NW = NC * NS
PER_W = TOTAL // NW
CH = 80
NFULL = PER_W // CH
NB = 5


def _sc_gather_kernel(table_hbm, idx_hbm, out_hbm, idx_v, rows_v, gsem, wsem):
    wid = lax.axis_index("s") * NC + lax.axis_index("c")
    base = pl.multiple_of(wid * PER_W, 16)
    pltpu.sync_copy(idx_hbm.at[pl.ds(base, PER_W)], idx_v)

    def group(i, carry):
        gathers = []
        for b in range(NB):
            off = pl.multiple_of((i * NB + b) * CH, CH)
            @pl.when(i >= 1)
            def _(off=off, b=b):
                prev = pl.multiple_of(off - NB * CH, CH)
                pltpu.make_async_copy(
                    rows_v.at[b], out_hbm.at[pl.ds(base + prev, CH)],
                    wsem.at[b]).wait()
            g = pltpu.make_async_copy(
                table_hbm.at[idx_v.at[pl.ds(off, CH)]], rows_v.at[b],
                gsem.at[b])
            g.start()
            gathers.append((off, b, g))
        for off, b, g in gathers:
            g.wait()
            pltpu.make_async_copy(
                rows_v.at[b], out_hbm.at[pl.ds(base + off, CH)],
                wsem.at[b]).start()
        return carry

    lax.fori_loop(0, NFULL // NB, group, 0, unroll=False)

    for b in range(NB):
        off = (NFULL - NB + b) * CH
        pltpu.make_async_copy(
            rows_v.at[b], out_hbm.at[pl.ds(base + off, CH)], wsem.at[b]).wait()


def _sc_gather(table, idx_flat):
    mesh = plsc.VectorSubcoreMesh(core_axis_name="c", subcore_axis_name="s")
    return pl.kernel(
        _sc_gather_kernel,
        out_type=jax.ShapeDtypeStruct((TOTAL, F), jnp.float32),
        mesh=mesh,
        scratch_types=[
            pltpu.VMEM((PER_W,), jnp.int32),
            pltpu.VMEM((NB, CH, F), jnp.float32),
            pltpu.SemaphoreType.DMA((NB,)),
            pltpu.SemaphoreType.DMA((NB,)),
        ],
    )(table, idx_flat)


BN = 1000
BL = 2000


def _ln_silu_body(x_ref, g_ref, b_ref, o_ref):
    x = x_ref[...]
    mu = jnp.mean(x, axis=-1, keepdims=True)
    var = jnp.mean((x - mu) ** 2, axis=-1, keepdims=True)
    t = (x - mu) / jnp.sqrt(var + 1e-5) * g_ref[...] + b_ref[...]
    o_ref[...] = t * jax.nn.sigmoid(t)


def _ln_silu(x, g, b):
    return pl.pallas_call(
        _ln_silu_body,
        grid=(N // BL,),
        in_specs=[
            pl.BlockSpec((BL, F), lambda i: (i, 0)),
            pl.BlockSpec((1, F), lambda i: (0, 0)),
            pl.BlockSpec((1, F), lambda i: (0, 0)),
        ],
        out_specs=pl.BlockSpec((BL, F), lambda i: (i, 0)),
        out_shape=jax.ShapeDtypeStruct((N, F), jnp.float32),
    )(x, g.reshape(1, F), b.reshape(1, F))


def _nh_dot(g_ref, w_ref):
    return jnp.dot(g_ref[...].reshape(BN, KF), w_ref[...].reshape(KF, F),
                   preferred_element_type=jnp.float32)


def _mm_ln_silu_body(g_ref, w_ref, b_ref, lg_ref, lb_ref, o_ref):
    y = _nh_dot(g_ref, w_ref) + b_ref[...]
    mu = jnp.mean(y, axis=-1, keepdims=True)
    var = jnp.mean((y - mu) ** 2, axis=-1, keepdims=True)
    t = (y - mu) / jnp.sqrt(var + 1e-5) * lg_ref[...] + lb_ref[...]
    o_ref[...] = t * jax.nn.sigmoid(t)


def _mm_ln_silu(gath, w, b, lg, lb):
    return pl.pallas_call(
        _mm_ln_silu_body,
        grid=(N // BN,),
        in_specs=[
            pl.BlockSpec((BN, K, F), lambda i: (i, 0, 0)),
            pl.BlockSpec((K, F, F), lambda i: (0, 0, 0)),
            pl.BlockSpec((1, F), lambda i: (0, 0)),
            pl.BlockSpec((1, F), lambda i: (0, 0)),
            pl.BlockSpec((1, F), lambda i: (0, 0)),
        ],
        out_specs=pl.BlockSpec((BN, F), lambda i: (i, 0)),
        out_shape=jax.ShapeDtypeStruct((N, F), jnp.float32),
    )(gath, w, b.reshape(1, F), lg.reshape(1, F), lb.reshape(1, F))


def _mm_res_body(g_ref, w_ref, b_ref, x_ref, o_ref):
    o_ref[...] = _nh_dot(g_ref, w_ref) + b_ref[...] + x_ref[...]


def _mm_res(gath, w, b, x):
    return pl.pallas_call(
        _mm_res_body,
        grid=(N // BN,),
        in_specs=[
            pl.BlockSpec((BN, K, F), lambda i: (i, 0, 0)),
            pl.BlockSpec((K, F, F), lambda i: (0, 0, 0)),
            pl.BlockSpec((1, F), lambda i: (0, 0)),
            pl.BlockSpec((BN, F), lambda i: (i, 0)),
        ],
        out_specs=pl.BlockSpec((BN, F), lambda i: (i, 0)),
        out_shape=jax.ShapeDtypeStruct((N, F), jnp.float32),
    )(gath, w, b.reshape(1, F), x)


def kernel(x, adjc, ln1_g, ln1_b, w1, b1, ln2_g, ln2_b, w2, b2):
    idx_flat = adjc.reshape(TOTAL)
    w1r = w1.reshape(K, F, F)
    w2r = w2.reshape(K, F, F)
    h1 = _ln_silu(x, ln1_g, ln1_b)
    g1 = _sc_gather(h1, idx_flat)
    h2 = _mm_ln_silu(g1.reshape(N, K, F), w1r, b1, ln2_g, ln2_b)
    g2 = _sc_gather(h2, idx_flat)
    return _mm_res(g2.reshape(N, K, F), w2r, b2, x)
